# baseline (speedup 1.0000x reference)
import functools

import jax
import jax.numpy as jnp
from jax import lax
from jax.experimental import pallas as pl
from jax.experimental.pallas import tpu as pltpu
from jax.experimental.pallas import tpu_sc as plsc

_N = 10000
_E2 = 640000
_D = 128
_H1 = 32
_H2 = 16
_NC = 2
_NS = 16
_T = 160
_B = 125

_MESH = plsc.VectorSubcoreMesh(core_axis_name="c", subcore_axis_name="s")
_F32 = jnp.float32
_SC_PARAMS = pltpu.CompilerParams(use_tc_tiling_on_sc=False)


_DW = 8


@functools.partial(
    pl.kernel,
    out_type=jax.ShapeDtypeStruct((_NC, _N, _DW), _F32),
    mesh=_MESH,
    compiler_params=_SC_PARAMS,
    scratch_types=[
        pltpu.VMEM((_T, _B), jnp.int32),
        pltpu.VMEM((_B, _DW), _F32),
        pltpu.VMEM_SHARED((_N, _DW), _F32),
    ],
)
def _sc_degree(dst_hbm, ones_hbm, zeros_hbm, out_hbm, dst_v, ones_v, acc):
    c = lax.axis_index("c")
    s = lax.axis_index("s")
    rows = 1000

    @pl.when(s < 10)
    def _():
        pltpu.sync_copy(zeros_hbm.at[pl.ds(s * rows, rows)],
                        acc.at[pl.ds(s * rows, rows)])

    pltpu.sync_copy(dst_hbm.at[c, s], dst_v)
    pltpu.sync_copy(ones_hbm, ones_v)
    plsc.subcore_barrier()

    def body(t, carry):
        pltpu.sync_copy(ones_v, acc.at[dst_v.at[t]], add=True)
        return carry

    lax.fori_loop(0, _T, body, 0)
    plsc.subcore_barrier()

    @pl.when(s < 10)
    def _():
        pltpu.sync_copy(acc.at[pl.ds(s * rows, rows)],
                        out_hbm.at[c, pl.ds(s * rows, rows)])


def _make_sc_pass(F):
    NBUF = 8

    @functools.partial(
        pl.kernel,
        out_type=jax.ShapeDtypeStruct((_NC, _N, F), _F32),
        mesh=_MESH,
        compiler_params=_SC_PARAMS,
        scratch_types=[
            pltpu.VMEM((_T, _B), jnp.int32),
            pltpu.VMEM((_T, _B), jnp.int32),
            [pltpu.VMEM((_B, F), _F32)] * NBUF,
            pltpu.VMEM_SHARED((_N, F), _F32),
            [pltpu.SemaphoreType.DMA] * NBUF,
            [pltpu.SemaphoreType.DMA] * NBUF,
        ],
    )
    def _sc_pass(g_hbm, src_hbm, dst_hbm, zeros_hbm, out_hbm,
                 src_v, dst_v, bufs, acc, gsems, ssems):
        c = lax.axis_index("c")
        s = lax.axis_index("s")
        rows = 1000

        @pl.when(s < 10)
        def _():
            pltpu.sync_copy(zeros_hbm.at[pl.ds(s * rows, rows)],
                            acc.at[pl.ds(s * rows, rows)])

        pltpu.sync_copy(src_hbm.at[c, s], src_v)
        pltpu.sync_copy(dst_hbm.at[c, s], dst_v)
        plsc.subcore_barrier()

        def gath(t, b):
            pltpu.async_copy(g_hbm.at[src_v.at[t]], bufs[b], gsems[b])

        def gath_wait(t, b):
            pltpu.make_async_copy(g_hbm.at[src_v.at[t]], bufs[b], gsems[b]).wait()

        def scat(t, b):
            pltpu.async_copy(bufs[b], acc.at[dst_v.at[t]], ssems[b], add=True)

        def scat_wait(t, b):
            pltpu.make_async_copy(bufs[b], acc.at[dst_v.at[t]], ssems[b]).wait()

        for j in range(NBUF):
            gath(j, j)

        def body(i, carry):
            t = i * NBUF
            for j in range(NBUF):
                gath_wait(t + j, j)
                scat(t + j, j)

            for j in range(NBUF):
                scat_wait(t + j, j)

                @pl.when(t + NBUF + j < _T)
                def _():
                    gath(t + NBUF + j, j)

            return carry

        lax.fori_loop(0, _T // NBUF, body, 0)
        plsc.subcore_barrier()

        @pl.when(s < 10)
        def _():
            pltpu.sync_copy(acc.at[pl.ds(s * rows, rows)],
                            out_hbm.at[c, pl.ds(s * rows, rows)])

    return _sc_pass


_sc_pass32 = _make_sc_pass(_H1)
_sc_pass16 = _make_sc_pass(_H2)


def _tc_a_body(degp, x, w1, dis_out, g1_out):
    deg = degp[0, :, 0:1] + degp[1, :, 0:1] + 1.0
    dis = lax.rsqrt(deg)
    h = lax.dot_general(x[...], w1[...], (((1,), (1,)), ((), ())),
                        preferred_element_type=_F32)
    dis_out[...] = dis
    g1_out[...] = h * dis


_tc_a = pl.pallas_call(
    _tc_a_body,
    out_shape=(jax.ShapeDtypeStruct((_N, 1), _F32),
               jax.ShapeDtypeStruct((_N, _H1), _F32)),
)


def _tc_b_body(p, g1, dis, b1, w2, g2_out):
    dis_v = dis[...]
    t = jnp.maximum((p[0] + p[1] + g1[...]) * dis_v + b1[...], 0.0)
    h2 = lax.dot_general(t, w2[...], (((1,), (1,)), ((), ())),
                         preferred_element_type=_F32)
    g2_out[...] = h2 * dis_v


_tc_b = pl.pallas_call(
    _tc_b_body,
    out_shape=jax.ShapeDtypeStruct((_N, _H2), _F32),
)


def _tc_c1_body(q, g2, dis, b2, z_out):
    z_out[...] = (q[0] + q[1] + g2[...]) * dis[...] + b2[...]


_tc_c1 = pl.pallas_call(
    _tc_c1_body,
    out_shape=jax.ShapeDtypeStruct((_N, _H2), _F32),
)


_BM = 256


def _tc_c2_body(zi, zf, out):
    out[...] = lax.dot_general(zi[...], zf[...], (((1,), (1,)), ((), ())),
                               preferred_element_type=_F32)


_tc_c2 = pl.pallas_call(
    _tc_c2_body,
    grid=(pl.cdiv(_N, _BM),),
    in_specs=[pl.BlockSpec((_BM, _H2), lambda i: (i, 0)),
              pl.BlockSpec((_N, _H2), lambda i: (0, 0))],

    out_specs=pl.BlockSpec((_BM, _N), lambda i: (i, 0)),
    out_shape=jax.ShapeDtypeStruct((_N, _N), _F32),
)


def kernel(x, edge_index, W1, b1, W2, b2):
    ei0 = edge_index[0]
    ei1 = edge_index[1]
    src = jnp.concatenate([ei0, ei1]).reshape(_NC, _NS, _T, _B)
    dst = jnp.concatenate([ei1, ei0]).reshape(_NC, _NS, _T, _B)
    ones = jnp.ones((_B, _DW), _F32)
    zeros1 = jnp.zeros((_N, _DW), _F32)
    zeros32 = jnp.zeros((_N, _H1), _F32)
    zeros16 = jnp.zeros((_N, _H2), _F32)

    degp = _sc_degree(dst, ones, zeros1)
    dis, g1 = _tc_a(degp, x, W1)
    p = _sc_pass32(g1, src, dst, zeros32)
    g2 = _tc_b(p, g1, dis, b1, W2)
    q = _sc_pass16(g2, src, dst, zeros16)
    z = _tc_c1(q, g2, dis, b2)
    return _tc_c2(z, z)


# speedup vs baseline: 1.0131x; 1.0131x over previous
import functools

import jax
import jax.numpy as jnp
from jax import lax
from jax.experimental import pallas as pl
from jax.experimental.pallas import tpu as pltpu
from jax.experimental.pallas import tpu_sc as plsc

_N = 10000
_E2 = 640000
_D = 128
_H1 = 32
_H2 = 16
_NC = 2
_NS = 16
_T = 160
_B = 125

_MESH = plsc.VectorSubcoreMesh(core_axis_name="c", subcore_axis_name="s")
_F32 = jnp.float32
_SC_PARAMS = pltpu.CompilerParams(use_tc_tiling_on_sc=False)


_DW = 8


@functools.partial(
    pl.kernel,
    out_type=jax.ShapeDtypeStruct((_NC, _N, _DW), _F32),
    mesh=_MESH,
    compiler_params=_SC_PARAMS,
    scratch_types=[
        pltpu.VMEM((_T, _B), jnp.int32),
        pltpu.VMEM((_B, _DW), _F32),
        pltpu.VMEM_SHARED((_N, _DW), _F32),
    ],
)
def _sc_degree(dst_hbm, ones_hbm, zeros_hbm, out_hbm, dst_v, ones_v, acc):
    c = lax.axis_index("c")
    s = lax.axis_index("s")
    rows = 1000

    @pl.when(s < 10)
    def _():
        pltpu.sync_copy(zeros_hbm.at[pl.ds(s * rows, rows)],
                        acc.at[pl.ds(s * rows, rows)])

    pltpu.sync_copy(dst_hbm.at[c, s], dst_v)
    pltpu.sync_copy(ones_hbm, ones_v)
    plsc.subcore_barrier()

    def body(t, carry):
        pltpu.sync_copy(ones_v, acc.at[dst_v.at[t]], add=True)
        return carry

    lax.fori_loop(0, _T, body, 0)
    plsc.subcore_barrier()

    @pl.when(s < 10)
    def _():
        pltpu.sync_copy(acc.at[pl.ds(s * rows, rows)],
                        out_hbm.at[c, pl.ds(s * rows, rows)])


def _make_sc_pass(F):
    NBUF = 8

    @functools.partial(
        pl.kernel,
        out_type=jax.ShapeDtypeStruct((_NC, _N, F), _F32),
        mesh=_MESH,
        compiler_params=_SC_PARAMS,
        scratch_types=[
            pltpu.VMEM((_T, _B), jnp.int32),
            pltpu.VMEM((_T, _B), jnp.int32),
            [pltpu.VMEM((_B, F), _F32)] * NBUF,
            pltpu.VMEM_SHARED((_N, F), _F32),
            [pltpu.SemaphoreType.DMA] * NBUF,
            [pltpu.SemaphoreType.DMA] * NBUF,
        ],
    )
    def _sc_pass(g_hbm, src_hbm, dst_hbm, zeros_hbm, out_hbm,
                 src_v, dst_v, bufs, acc, gsems, ssems):
        c = lax.axis_index("c")
        s = lax.axis_index("s")
        rows = 1000

        @pl.when(s < 10)
        def _():
            pltpu.sync_copy(zeros_hbm.at[pl.ds(s * rows, rows)],
                            acc.at[pl.ds(s * rows, rows)])

        pltpu.sync_copy(src_hbm.at[c, s], src_v)
        pltpu.sync_copy(dst_hbm.at[c, s], dst_v)
        plsc.subcore_barrier()

        def gath(t, b):
            pltpu.async_copy(g_hbm.at[src_v.at[t]], bufs[b], gsems[b])

        def gath_wait(t, b):
            pltpu.make_async_copy(g_hbm.at[src_v.at[t]], bufs[b], gsems[b]).wait()

        def scat(t, b):
            pltpu.async_copy(bufs[b], acc.at[dst_v.at[t]], ssems[b], add=True)

        def scat_wait(t, b):
            pltpu.make_async_copy(bufs[b], acc.at[dst_v.at[t]], ssems[b]).wait()

        for j in range(NBUF):
            gath(j, j)

        def body(i, carry):
            t = i * NBUF
            for j in range(NBUF):
                gath_wait(t + j, j)
                pltpu.sync_copy(bufs[j], acc.at[dst_v.at[t + j]], add=True)

                @pl.when(t + NBUF + j < _T)
                def _():
                    gath(t + NBUF + j, j)

            return carry

        lax.fori_loop(0, _T // NBUF, body, 0)
        plsc.subcore_barrier()

        @pl.when(s < 10)
        def _():
            pltpu.sync_copy(acc.at[pl.ds(s * rows, rows)],
                            out_hbm.at[c, pl.ds(s * rows, rows)])

    return _sc_pass


_sc_pass32 = _make_sc_pass(_H1)
_sc_pass16 = _make_sc_pass(_H2)


def _tc_a_body(degp, x, w1, dis_out, g1_out):
    deg = degp[0, :, 0:1] + degp[1, :, 0:1] + 1.0
    dis = lax.rsqrt(deg)
    h = lax.dot_general(x[...], w1[...], (((1,), (1,)), ((), ())),
                        preferred_element_type=_F32)
    dis_out[...] = dis
    g1_out[...] = h * dis


_tc_a = pl.pallas_call(
    _tc_a_body,
    out_shape=(jax.ShapeDtypeStruct((_N, 1), _F32),
               jax.ShapeDtypeStruct((_N, _H1), _F32)),
)


def _tc_b_body(p, g1, dis, b1, w2, g2_out):
    dis_v = dis[...]
    t = jnp.maximum((p[0] + p[1] + g1[...]) * dis_v + b1[...], 0.0)
    h2 = lax.dot_general(t, w2[...], (((1,), (1,)), ((), ())),
                         preferred_element_type=_F32)
    g2_out[...] = h2 * dis_v


_tc_b = pl.pallas_call(
    _tc_b_body,
    out_shape=jax.ShapeDtypeStruct((_N, _H2), _F32),
)


def _tc_c1_body(q, g2, dis, b2, z_out):
    z_out[...] = (q[0] + q[1] + g2[...]) * dis[...] + b2[...]


_tc_c1 = pl.pallas_call(
    _tc_c1_body,
    out_shape=jax.ShapeDtypeStruct((_N, _H2), _F32),
)


_BM = 256


def _tc_c2_body(zi, zf, out):
    out[...] = lax.dot_general(zi[...], zf[...], (((1,), (1,)), ((), ())),
                               preferred_element_type=_F32)


_tc_c2 = pl.pallas_call(
    _tc_c2_body,
    grid=(pl.cdiv(_N, _BM),),
    in_specs=[pl.BlockSpec((_BM, _H2), lambda i: (i, 0)),
              pl.BlockSpec((_N, _H2), lambda i: (0, 0))],

    out_specs=pl.BlockSpec((_BM, _N), lambda i: (i, 0)),
    out_shape=jax.ShapeDtypeStruct((_N, _N), _F32),
)


def kernel(x, edge_index, W1, b1, W2, b2):
    ei0 = edge_index[0]
    ei1 = edge_index[1]
    src = jnp.concatenate([ei0, ei1]).reshape(_NC, _NS, _T, _B)
    dst = jnp.concatenate([ei1, ei0]).reshape(_NC, _NS, _T, _B)
    ones = jnp.ones((_B, _DW), _F32)
    zeros1 = jnp.zeros((_N, _DW), _F32)
    zeros32 = jnp.zeros((_N, _H1), _F32)
    zeros16 = jnp.zeros((_N, _H2), _F32)

    degp = _sc_degree(dst, ones, zeros1)
    dis, g1 = _tc_a(degp, x, W1)
    p = _sc_pass32(g1, src, dst, zeros32)
    g2 = _tc_b(p, g1, dis, b1, W2)
    q = _sc_pass16(g2, src, dst, zeros16)
    z = _tc_c1(q, g2, dis, b2)
    return _tc_c2(z, z)


# speedup vs baseline: 1.0486x; 1.0350x over previous
import functools

import jax
import jax.numpy as jnp
from jax import lax
from jax.experimental import pallas as pl
from jax.experimental.pallas import tpu as pltpu
from jax.experimental.pallas import tpu_sc as plsc

_N = 10000
_E2 = 640000
_D = 128
_H1 = 32
_H2 = 16
_NC = 2
_NS = 16
_T = 160
_B = 125

_MESH = plsc.VectorSubcoreMesh(core_axis_name="c", subcore_axis_name="s")
_F32 = jnp.float32
_SC_PARAMS = pltpu.CompilerParams(use_tc_tiling_on_sc=False)


_DW = 8


@functools.partial(
    pl.kernel,
    out_type=jax.ShapeDtypeStruct((_NC, _N, _DW), _F32),
    mesh=_MESH,
    compiler_params=_SC_PARAMS,
    scratch_types=[
        pltpu.VMEM((_T, _B), jnp.int32),
        pltpu.VMEM((_B, _DW), _F32),
        pltpu.VMEM_SHARED((_N, _DW), _F32),
    ],
)
def _sc_degree(ei_hbm, ones_hbm, zeros_hbm, out_hbm, dst_v, ones_v, acc):
    c = lax.axis_index("c")
    s = lax.axis_index("s")
    rows = 1000

    @pl.when(s < 10)
    def _():
        pltpu.sync_copy(zeros_hbm.at[pl.ds(s * rows, rows)],
                        acc.at[pl.ds(s * rows, rows)])

    pltpu.sync_copy(ei_hbm.at[1 - c, s], dst_v)
    pltpu.sync_copy(ones_hbm, ones_v)
    plsc.subcore_barrier()

    def body(t, carry):
        pltpu.sync_copy(ones_v, acc.at[dst_v.at[t]], add=True)
        return carry

    lax.fori_loop(0, _T, body, 0)
    plsc.subcore_barrier()

    @pl.when(s < 10)
    def _():
        pltpu.sync_copy(acc.at[pl.ds(s * rows, rows)],
                        out_hbm.at[c, pl.ds(s * rows, rows)])


def _make_sc_pass(F):
    NBUF = 8

    @functools.partial(
        pl.kernel,
        out_type=jax.ShapeDtypeStruct((_NC, _N, F), _F32),
        mesh=_MESH,
        compiler_params=_SC_PARAMS,
        scratch_types=[
            pltpu.VMEM((_T, _B), jnp.int32),
            pltpu.VMEM((_T, _B), jnp.int32),
            [pltpu.VMEM((_B, F), _F32)] * NBUF,
            pltpu.VMEM_SHARED((_N, F), _F32),
            [pltpu.SemaphoreType.DMA] * NBUF,
        ],
    )
    def _sc_pass(g_hbm, ei_hbm, zeros_hbm, out_hbm,
                 src_v, dst_v, bufs, acc, gsems):
        c = lax.axis_index("c")
        s = lax.axis_index("s")
        rows = 1000

        @pl.when(s < 10)
        def _():
            pltpu.sync_copy(zeros_hbm.at[pl.ds(s * rows, rows)],
                            acc.at[pl.ds(s * rows, rows)])

        pltpu.sync_copy(ei_hbm.at[c, s], src_v)
        pltpu.sync_copy(ei_hbm.at[1 - c, s], dst_v)
        plsc.subcore_barrier()

        def gath(t, b):
            pltpu.async_copy(g_hbm.at[src_v.at[t]], bufs[b], gsems[b])

        def gath_wait(t, b):
            pltpu.make_async_copy(g_hbm.at[src_v.at[t]], bufs[b], gsems[b]).wait()

        for j in range(NBUF):
            gath(j, j)

        def body(i, carry):
            t = i * NBUF
            for j in range(NBUF):
                gath_wait(t + j, j)
                pltpu.sync_copy(bufs[j], acc.at[dst_v.at[t + j]], add=True)

                @pl.when(t + NBUF + j < _T)
                def _():
                    gath(t + NBUF + j, j)

            return carry

        lax.fori_loop(0, _T // NBUF, body, 0)
        plsc.subcore_barrier()

        @pl.when(s < 10)
        def _():
            pltpu.sync_copy(acc.at[pl.ds(s * rows, rows)],
                            out_hbm.at[c, pl.ds(s * rows, rows)])

    return _sc_pass


_sc_pass32 = _make_sc_pass(_H1)
_sc_pass16 = _make_sc_pass(_H2)


def _tc_a_body(degp, x, w1, dis_out, g1_out):
    deg = degp[0, :, 0:1] + degp[1, :, 0:1] + 1.0
    dis = lax.rsqrt(deg)
    h = lax.dot_general(x[...], w1[...], (((1,), (1,)), ((), ())),
                        preferred_element_type=_F32)
    dis_out[...] = dis
    g1_out[...] = h * dis


_tc_a = pl.pallas_call(
    _tc_a_body,
    out_shape=(jax.ShapeDtypeStruct((_N, 1), _F32),
               jax.ShapeDtypeStruct((_N, _H1), _F32)),
)


def _tc_b_body(p, g1, dis, b1, w2, g2_out):
    dis_v = dis[...]
    t = jnp.maximum((p[0] + p[1] + g1[...]) * dis_v + b1[...], 0.0)
    h2 = lax.dot_general(t, w2[...], (((1,), (1,)), ((), ())),
                         preferred_element_type=_F32)
    g2_out[...] = h2 * dis_v


_tc_b = pl.pallas_call(
    _tc_b_body,
    out_shape=jax.ShapeDtypeStruct((_N, _H2), _F32),
)


def _tc_c1_body(q, g2, dis, b2, z_out):
    z_out[...] = (q[0] + q[1] + g2[...]) * dis[...] + b2[...]


_tc_c1 = pl.pallas_call(
    _tc_c1_body,
    out_shape=jax.ShapeDtypeStruct((_N, _H2), _F32),
)


_BM = 256


def _tc_c2_body(zi, zf, out):
    out[...] = lax.dot_general(zi[...], zf[...], (((1,), (1,)), ((), ())),
                               preferred_element_type=_F32)


_tc_c2 = pl.pallas_call(
    _tc_c2_body,
    grid=(pl.cdiv(_N, _BM),),
    in_specs=[pl.BlockSpec((_BM, _H2), lambda i: (i, 0)),
              pl.BlockSpec((_N, _H2), lambda i: (0, 0))],

    out_specs=pl.BlockSpec((_BM, _N), lambda i: (i, 0)),
    out_shape=jax.ShapeDtypeStruct((_N, _N), _F32),
)


def kernel(x, edge_index, W1, b1, W2, b2):
    ei_r = edge_index.reshape(2, _NS, _T, _B)
    ones = jnp.ones((_B, _DW), _F32)
    zeros1 = jnp.zeros((_N, _DW), _F32)
    zeros32 = jnp.zeros((_N, _H1), _F32)
    zeros16 = jnp.zeros((_N, _H2), _F32)

    degp = _sc_degree(ei_r, ones, zeros1)
    dis, g1 = _tc_a(degp, x, W1)
    p = _sc_pass32(g1, ei_r, zeros32)
    g2 = _tc_b(p, g1, dis, b1, W2)
    q = _sc_pass16(g2, ei_r, zeros16)
    z = _tc_c1(q, g2, dis, b2)
    return _tc_c2(z, z)
